# Initial kernel scaffold; baseline (speedup 1.0000x reference)
import jax
import jax.numpy as jnp
from jax.experimental import pallas as pl


def kernel(x, edge_index, batch, W1_0, b1_0, g1_0, be1_0, W2_0, b2_0, eps_0, bng_0, bnb_0, W1_1, b1_1, g1_1, be1_1, W2_1, b2_1, eps_1, bng_1, bnb_1, W1_2, b1_2, g1_2, be1_2, W2_2, b2_2, eps_2, bng_2, bnb_2, fW1, fb1, fg, fbe, fW2, fb2):
    raise NotImplementedError("write your pallas kernel here")


# speedup vs baseline: 4.6128x; 4.6128x over previous
import functools

import jax
import jax.numpy as jnp
import numpy as np
from jax import lax
from jax.experimental import pallas as pl
from jax.experimental.pallas import tpu as pltpu
from jax.experimental.pallas import tpu_sc as plsc

N, E, D, H, O, G = 10000, 320000, 128, 128, 128, 64
BN_S = 1.0 / np.sqrt(1.0 + 1e-5)

NC, NS = 2, 16
NW = NC * NS
EPW = E // NW
K = 80
NCH = EPW // K
ZR = 80
NB = N // ZR


def _agg_body(h_hbm, src_hbm, dst_hbm, out_hbm,
              src_v, dst_v, rows_v, zbuf_v, acc_sh, sem):
    c = lax.axis_index("c")
    s = lax.axis_index("s")
    wid = s * NC + c

    def zstore(i, _):
        zbuf_v[i // 8, pl.ds((i % 8) * 16, 16)] = jnp.zeros((16,), jnp.float32)
        return 0
    lax.fori_loop(0, ZR * 8, zstore, 0)

    def zblk(j, _):
        b = s + j * NS

        @pl.when(b < NB)
        def _():
            pltpu.sync_copy(zbuf_v, acc_sh.at[pl.ds(b * ZR, ZR)])
        return 0
    lax.fori_loop(0, (NB + NS - 1) // NS, zblk, 0)
    plsc.subcore_barrier()

    def body(g, _):
        off = wid * EPW + g * K
        pltpu.sync_copy(src_hbm.at[pl.ds(off, K)], src_v)
        pltpu.sync_copy(dst_hbm.at[pl.ds(off, K)], dst_v)
        pltpu.async_copy(h_hbm.at[src_v], rows_v, sem).wait()
        pltpu.sync_copy(rows_v, acc_sh.at[dst_v], add=True)
        return 0
    lax.fori_loop(0, NCH, body, 0)
    plsc.subcore_barrier()

    def wblk(j, _):
        b = s + j * NS

        @pl.when(b < NB)
        def _():
            pltpu.sync_copy(acc_sh.at[pl.ds(b * ZR, ZR)],
                            out_hbm.at[c, pl.ds(b * ZR, ZR)])
        return 0
    lax.fori_loop(0, (NB + NS - 1) // NS, wblk, 0)


_agg_call = functools.partial(
    pl.kernel,
    out_type=jax.ShapeDtypeStruct((NC, N, H), jnp.float32),
    mesh=plsc.VectorSubcoreMesh(core_axis_name="c", subcore_axis_name="s",
                                num_cores=NC, num_subcores=NS),
    scratch_types=[
        pltpu.VMEM((K,), jnp.int32),
        pltpu.VMEM((K,), jnp.int32),
        pltpu.VMEM((K, H), jnp.float32),
        pltpu.VMEM((ZR, H), jnp.float32),
        pltpu.VMEM_SHARED((N, H), jnp.float32),
        pltpu.SemaphoreType.DMA,
    ],
)(_agg_body)


BM = 1000


def _mlp_body(eps_ref, h_ref, parts_ref, W1_ref, b1_ref, g1_ref, be_ref,
              W2_ref, b2_ref, bng_ref, bnb_ref, out_ref):
    z = h_ref[...] * (1.0 + eps_ref[0, 0]) + parts_ref[0] + parts_ref[1]
    t = jnp.dot(z, W1_ref[...], preferred_element_type=jnp.float32) + b1_ref[...]
    t = jnp.maximum(t * (g1_ref[...] * BN_S) + be_ref[...], 0.0)
    y = jnp.dot(t, W2_ref[...], preferred_element_type=jnp.float32) + b2_ref[...]
    out_ref[...] = jnp.maximum(y * (bng_ref[...] * BN_S) + bnb_ref[...], 0.0)


def _mlp_call(eps, h, parts, W1, b1, g1, be, W2, b2, bng, bnb):
    vspec = pl.BlockSpec((1, H), lambda i: (0, 0))
    wspec = pl.BlockSpec((H, H), lambda i: (0, 0))
    return pl.pallas_call(
        _mlp_body,
        grid=(N // BM,),
        in_specs=[
            pl.BlockSpec(memory_space=pltpu.SMEM),
            pl.BlockSpec((BM, H), lambda i: (i, 0)),
            pl.BlockSpec((NC, BM, H), lambda i: (0, i, 0)),
            wspec, vspec, vspec, vspec, wspec, vspec, vspec, vspec,
        ],
        out_specs=pl.BlockSpec((BM, H), lambda i: (i, 0)),
        out_shape=jax.ShapeDtypeStruct((N, H), jnp.float32),
    )(eps.reshape(1, 1), h, parts,
      W1, b1.reshape(1, H), g1.reshape(1, H), be.reshape(1, H),
      W2, b2.reshape(1, H), bng.reshape(1, H), bnb.reshape(1, H))


def _pool_body(batch_ref, h_ref, fW1_ref, fb1_ref, fg_ref, fbe_ref,
               fW2_ref, fb2_ref, out_ref, acc_ref):
    i = pl.program_id(0)

    @pl.when(i == 0)
    def _():
        acc_ref[...] = jnp.zeros((G, H), jnp.float32)

    seg = batch_ref[0]
    onehot = (lax.broadcasted_iota(jnp.int32, (G, BM), 0) == seg).astype(jnp.float32)
    acc_ref[...] += jnp.dot(onehot, h_ref[...], preferred_element_type=jnp.float32)

    @pl.when(i == N // BM - 1)
    def _():
        z = acc_ref[...]
        t = jnp.dot(z, fW1_ref[...], preferred_element_type=jnp.float32) + fb1_ref[...]
        t = jnp.maximum(t * (fg_ref[...] * BN_S) + fbe_ref[...], 0.0)
        out_ref[...] = (jnp.dot(t, fW2_ref[...], preferred_element_type=jnp.float32)
                        + fb2_ref[...])


def _pool_call(batch3d, h, fW1, fb1, fg, fbe, fW2, fb2):
    vspec = pl.BlockSpec((1, H), lambda i: (0, 0))
    return pl.pallas_call(
        _pool_body,
        grid=(N // BM,),
        in_specs=[
            pl.BlockSpec((1, 1, BM), lambda i: (i, 0, 0)),
            pl.BlockSpec((BM, H), lambda i: (i, 0)),
            pl.BlockSpec((H, H), lambda i: (0, 0)),
            vspec, vspec, vspec,
            pl.BlockSpec((H, O), lambda i: (0, 0)),
            pl.BlockSpec((1, O), lambda i: (0, 0)),
        ],
        out_specs=pl.BlockSpec((G, O), lambda i: (0, 0)),
        out_shape=jax.ShapeDtypeStruct((G, O), jnp.float32),
        scratch_shapes=[pltpu.VMEM((G, H), jnp.float32)],
    )(batch3d, h, fW1, fb1.reshape(1, H), fg.reshape(1, H), fbe.reshape(1, H),
      fW2, fb2.reshape(1, O))


def kernel(x, edge_index, batch,
           W1_0, b1_0, g1_0, be1_0, W2_0, b2_0, eps_0, bng_0, bnb_0,
           W1_1, b1_1, g1_1, be1_1, W2_1, b2_1, eps_1, bng_1, bnb_1,
           W1_2, b1_2, g1_2, be1_2, W2_2, b2_2, eps_2, bng_2, bnb_2,
           fW1, fb1, fg, fbe, fW2, fb2):
    src = edge_index[0]
    dst = edge_index[1]
    batch3d = batch.reshape(N // BM, 1, BM)
    layers = [
        (W1_0, b1_0, g1_0, be1_0, W2_0, b2_0, eps_0, bng_0, bnb_0),
        (W1_1, b1_1, g1_1, be1_1, W2_1, b2_1, eps_1, bng_1, bnb_1),
        (W1_2, b1_2, g1_2, be1_2, W2_2, b2_2, eps_2, bng_2, bnb_2),
    ]
    h = x
    for (W1, b1, g1, be, W2, b2, eps, bng, bnb) in layers:
        parts = _agg_call(h, src, dst)
        h = _mlp_call(eps, h, parts, W1, b1, g1, be, W2, b2, bng, bnb)
    return _pool_call(batch3d, h, fW1, fb1, fg, fbe, fW2, fb2)
